# baseline (speedup 1.0000x reference)
import functools

import jax
import jax.numpy as jnp
import numpy as np
from jax import lax
from jax.experimental import pallas as pl
from jax.experimental.pallas import tpu as pltpu
from jax.experimental.pallas import tpu_sc as plsc

NUM_CLASSES = 4096
MIX_NUM = 2
PATCH = 16
SMOOTH = 0.1
B = 256
OFF = SMOOTH / NUM_CLASSES
ON1 = (1.0 - SMOOTH) / MIX_NUM + OFF
ON2 = ((1.0 - SMOOTH) * np.array([0.5, 1.0, 0.5]) / MIX_NUM + OFF).astype(np.float32)


_SEL_BITS = (
    "1100000001001110010001010111100001001010110000111001100101011010"
    "1010100101111001000010111101010101101001111101000101100110101111"
    "1001111110000101011001110010001011101110101110000000111000010011"
    "0110"
)


def _compute_patch_mask():
    sel = np.array([int(ch) for ch in _SEL_BITS]).reshape(14, 14)
    mask = np.repeat(np.repeat(sel, PATCH, 0), PATCH, 1)
    return mask.astype(np.float32)


_PATCH_MASK = _compute_patch_mask()


ROWS = 16


def _mix_body(xa_ref, xb_ref, mask_ref, o_ref):
    m = mask_ref[...] != 0.0
    o_ref[0 : ROWS - 1] = jnp.where(m, xa_ref[1:ROWS], xa_ref[0 : ROWS - 1])
    o_ref[ROWS - 1 : ROWS] = jnp.where(m, xb_ref[...], xa_ref[ROWS - 1 : ROWS])


RPW = B // 32
TSTRIDE = 272


def _onehot_sc_body(t3_ref, o1_ref, o2_ref, t3_v, blk_v):
    cid = lax.axis_index("c")
    sid = lax.axis_index("s")
    wid = sid * 2 + cid
    base = wid * RPW

    pltpu.sync_copy(t3_ref, t3_v)

    offv = jnp.full((16,), OFF, jnp.float32)

    def fill_body(i, _):
        o = i * 128
        for k in range(8):
            blk_v[pl.ds(o + k * 16, 16)] = offv
        return _

    lax.fori_loop(0, RPW * NUM_CLASSES // 128, fill_body, None)

    rowv = jnp.arange(16, dtype=jnp.int32)
    lane_ok = rowv < RPW
    tvec = [t3_v[pl.ds(d * TSTRIDE + base, 16)] for d in range(3)]
    flat = [rowv * NUM_CLASSES + tvec[d] for d in range(3)]

    on1v = jnp.full((16,), ON1, jnp.float32)
    plsc.store_scatter(blk_v, [flat[1]], on1v, mask=lane_ok)
    plsc.store_scatter(blk_v, [flat[2]], on1v, mask=lane_ok)
    pltpu.sync_copy(blk_v, o1_ref.at[pl.ds(base * NUM_CLASSES, RPW * NUM_CLASSES)])

    plsc.store_scatter(blk_v, [flat[1]], offv, mask=lane_ok)
    plsc.store_scatter(blk_v, [flat[2]], offv, mask=lane_ok)
    for d in range(3):
        plsc.store_scatter(
            blk_v, [flat[d]], jnp.full((16,), float(ON2[d]), jnp.float32),
            mask=lane_ok,
        )
    pltpu.sync_copy(blk_v, o2_ref.at[pl.ds(base * NUM_CLASSES, RPW * NUM_CLASSES)])


_onehot_sc = functools.partial(
    pl.kernel,
    out_type=[
        jax.ShapeDtypeStruct((B * NUM_CLASSES,), jnp.float32),
        jax.ShapeDtypeStruct((B * NUM_CLASSES,), jnp.float32),
    ],
    mesh=plsc.VectorSubcoreMesh(core_axis_name="c", subcore_axis_name="s"),
    compiler_params=pltpu.CompilerParams(needs_layout_passes=False),
    scratch_types=[
        pltpu.VMEM((3 * TSTRIDE,), jnp.int32),
        pltpu.VMEM((RPW * NUM_CLASSES,), jnp.float32),
    ],
)(_onehot_sc_body)


def kernel(x, target):
    b, c, h, w = x.shape
    n = c * h * w
    sub, lane = n // 128, 128
    xf = x.reshape(b, sub, lane)
    mask = jnp.asarray(
        np.tile(_PATCH_MASK.reshape(-1), c).reshape(1, sub, lane)
    )

    t3f = (
        jnp.zeros((3, TSTRIDE), jnp.int32)
        .at[:, :b]
        .set(jnp.stack([jnp.roll(target, 1), target, jnp.roll(target, -1)]))
        .reshape(-1)
    )

    x_out = pl.pallas_call(
        _mix_body,
        grid=(b // ROWS,),
        in_specs=[
            pl.BlockSpec((ROWS, sub, lane), lambda t: (t, 0, 0)),
            pl.BlockSpec((1, sub, lane), lambda t: (((t + 1) * ROWS) % b, 0, 0)),
            pl.BlockSpec((1, sub, lane), lambda t: (0, 0, 0)),
        ],
        out_specs=pl.BlockSpec((ROWS, sub, lane), lambda t: (t, 0, 0)),
        out_shape=jax.ShapeDtypeStruct((b, sub, lane), x.dtype),
    )(xf, xf, mask).reshape(b, c, h, w)

    m2o_flat, m2m_flat = _onehot_sc(t3f)
    m2o_target = m2o_flat.reshape(b, NUM_CLASSES)
    m2m_target = m2m_flat.reshape(b, NUM_CLASSES)

    return (x_out, m2o_target, m2m_target)


# speedup vs baseline: 1.0128x; 1.0128x over previous
import functools

import jax
import jax.numpy as jnp
import numpy as np
from jax import lax
from jax.experimental import pallas as pl
from jax.experimental.pallas import tpu as pltpu
from jax.experimental.pallas import tpu_sc as plsc

NUM_CLASSES = 4096
MIX_NUM = 2
PATCH = 16
SMOOTH = 0.1
B = 256
OFF = SMOOTH / NUM_CLASSES
ON1 = (1.0 - SMOOTH) / MIX_NUM + OFF
ON2 = ((1.0 - SMOOTH) * np.array([0.5, 1.0, 0.5]) / MIX_NUM + OFF).astype(np.float32)


_SEL_BITS = (
    "1100000001001110010001010111100001001010110000111001100101011010"
    "1010100101111001000010111101010101101001111101000101100110101111"
    "1001111110000101011001110010001011101110101110000000111000010011"
    "0110"
)


def _compute_patch_mask():
    sel = np.array([int(ch) for ch in _SEL_BITS]).reshape(14, 14)
    mask = np.repeat(np.repeat(sel, PATCH, 0), PATCH, 1)
    return mask.astype(np.float32)


_PATCH_MASK = _compute_patch_mask()


ROWS = 16


def _mix_body(xa_ref, xb_ref, mask_ref, o_ref):
    m = mask_ref[...] != 0.0
    o_ref[0 : ROWS - 1] = jnp.where(m, xa_ref[1:ROWS], xa_ref[0 : ROWS - 1])
    o_ref[ROWS - 1 : ROWS] = jnp.where(m, xb_ref[...], xa_ref[ROWS - 1 : ROWS])


RPW = B // 32


def _onehot_sc_body(t_ref, o1_ref, o2_ref, t_v, blk_v):
    cid = lax.axis_index("c")
    sid = lax.axis_index("s")
    wid = sid * 2 + cid
    base = wid * RPW

    pltpu.sync_copy(t_ref.at[pl.ds(B - 8, 8)], t_v.at[pl.ds(0, 8)])
    pltpu.sync_copy(t_ref, t_v.at[pl.ds(8, B)])
    pltpu.sync_copy(t_ref.at[pl.ds(0, 16)], t_v.at[pl.ds(8 + B, 16)])

    offv = jnp.full((16,), OFF, jnp.float32)

    def fill_body(i, carry):
        o = i * 128
        for k in range(8):
            blk_v[pl.ds(o + k * 16, 16)] = offv
        return carry

    lax.fori_loop(0, RPW * NUM_CLASSES // 128, fill_body, None)

    rowv = jnp.arange(16, dtype=jnp.int32)
    lane_ok = rowv < RPW
    t0vec = t_v[pl.ds(base + 8, 16)]
    tmvec = t_v[pl.ds(base, 16)]
    def _lane_gather(vec, idx):
        return lax.gather(
            vec,
            idx[:, None],
            lax.GatherDimensionNumbers(
                offset_dims=(), collapsed_slice_dims=(0,), start_index_map=(0,)
            ),
            slice_sizes=(1,),
            mode=lax.GatherScatterMode.PROMISE_IN_BOUNDS,
        )

    tm1 = _lane_gather(tmvec, jnp.minimum(rowv + 7, 15))
    tp1 = _lane_gather(t0vec, jnp.minimum(rowv + 1, 15))
    tvec = [tm1, t0vec, tp1]
    flat = [rowv * NUM_CLASSES + tvec[d] for d in range(3)]

    on1v = jnp.full((16,), ON1, jnp.float32)
    plsc.store_scatter(blk_v, [flat[1]], on1v, mask=lane_ok)
    plsc.store_scatter(blk_v, [flat[2]], on1v, mask=lane_ok)
    pltpu.sync_copy(blk_v, o1_ref.at[pl.ds(base * NUM_CLASSES, RPW * NUM_CLASSES)])

    plsc.store_scatter(blk_v, [flat[1]], offv, mask=lane_ok)
    plsc.store_scatter(blk_v, [flat[2]], offv, mask=lane_ok)
    for d in range(3):
        plsc.store_scatter(
            blk_v, [flat[d]], jnp.full((16,), float(ON2[d]), jnp.float32),
            mask=lane_ok,
        )
    pltpu.sync_copy(blk_v, o2_ref.at[pl.ds(base * NUM_CLASSES, RPW * NUM_CLASSES)])


_onehot_sc = functools.partial(
    pl.kernel,
    out_type=[
        jax.ShapeDtypeStruct((B * NUM_CLASSES,), jnp.float32),
        jax.ShapeDtypeStruct((B * NUM_CLASSES,), jnp.float32),
    ],
    mesh=plsc.VectorSubcoreMesh(core_axis_name="c", subcore_axis_name="s"),
    compiler_params=pltpu.CompilerParams(needs_layout_passes=False),
    scratch_types=[
        pltpu.VMEM((8 + B + 16,), jnp.int32),
        pltpu.VMEM((RPW * NUM_CLASSES,), jnp.float32),
    ],
)(_onehot_sc_body)


def kernel(x, target):
    b, c, h, w = x.shape
    n = c * h * w
    sub, lane = n // 128, 128
    xf = x.reshape(b, sub, lane)
    mask = jnp.asarray(
        np.tile(_PATCH_MASK.reshape(-1), c).reshape(1, sub, lane)
    )

    x_out = pl.pallas_call(
        _mix_body,
        grid=(b // ROWS,),
        in_specs=[
            pl.BlockSpec((ROWS, sub, lane), lambda t: (t, 0, 0)),
            pl.BlockSpec((1, sub, lane), lambda t: (((t + 1) * ROWS) % b, 0, 0)),
            pl.BlockSpec((1, sub, lane), lambda t: (0, 0, 0)),
        ],
        out_specs=pl.BlockSpec((ROWS, sub, lane), lambda t: (t, 0, 0)),
        out_shape=jax.ShapeDtypeStruct((b, sub, lane), x.dtype),
    )(xf, xf, mask).reshape(b, c, h, w)

    m2o_flat, m2m_flat = _onehot_sc(target.astype(jnp.int32))
    m2o_target = m2o_flat.reshape(b, NUM_CLASSES)
    m2m_target = m2m_flat.reshape(b, NUM_CLASSES)

    return (x_out, m2o_target, m2m_target)


# speedup vs baseline: 1.0141x; 1.0013x over previous
import functools

import jax
import jax.numpy as jnp
import numpy as np
from jax import lax
from jax.experimental import pallas as pl
from jax.experimental.pallas import tpu as pltpu
from jax.experimental.pallas import tpu_sc as plsc

NUM_CLASSES = 4096
MIX_NUM = 2
PATCH = 16
SMOOTH = 0.1
B = 256
OFF = SMOOTH / NUM_CLASSES
ON1 = (1.0 - SMOOTH) / MIX_NUM + OFF
ON2 = ((1.0 - SMOOTH) * np.array([0.5, 1.0, 0.5]) / MIX_NUM + OFF).astype(np.float32)


_SEL_BITS = (
    "1100000001001110010001010111100001001010110000111001100101011010"
    "1010100101111001000010111101010101101001111101000101100110101111"
    "1001111110000101011001110010001011101110101110000000111000010011"
    "0110"
)


def _compute_patch_mask():
    sel = np.array([int(ch) for ch in _SEL_BITS]).reshape(14, 14)
    mask = np.repeat(np.repeat(sel, PATCH, 0), PATCH, 1)
    return mask.astype(np.float32)


_PATCH_MASK = _compute_patch_mask()


ROWS = 16


def _mix_body(xa_ref, xb_ref, mask_ref, o_ref):
    m = mask_ref[...] != 0.0
    o_ref[0 : ROWS - 1] = jnp.where(m, xa_ref[1:ROWS], xa_ref[0 : ROWS - 1])
    o_ref[ROWS - 1 : ROWS] = jnp.where(m, xb_ref[...], xa_ref[ROWS - 1 : ROWS])


RPW = B // 32


def _onehot_sc_body(t_ref, o1_ref, o2_ref, t_v, blk_v):
    cid = lax.axis_index("c")
    sid = lax.axis_index("s")
    wid = sid * 2 + cid
    base = wid * RPW

    pltpu.sync_copy(t_ref.at[pl.ds(B - 8, 8)], t_v.at[pl.ds(0, 8)])
    pltpu.sync_copy(t_ref, t_v.at[pl.ds(8, B)])
    pltpu.sync_copy(t_ref.at[pl.ds(0, 16)], t_v.at[pl.ds(8 + B, 16)])

    offv = jnp.full((16,), OFF, jnp.float32)

    def fill_body(i, carry):
        o = i * 128
        for k in range(8):
            blk_v[pl.ds(o + k * 16, 16)] = offv
        return carry

    lax.fori_loop(0, RPW * NUM_CLASSES // 128, fill_body, None)

    rowv = jnp.arange(16, dtype=jnp.int32)
    lane_ok = rowv < RPW
    t0vec = t_v[pl.ds(base + 8, 16)]
    tmvec = t_v[pl.ds(base, 16)]
    def _lane_gather(vec, idx):
        return lax.gather(
            vec,
            idx[:, None],
            lax.GatherDimensionNumbers(
                offset_dims=(), collapsed_slice_dims=(0,), start_index_map=(0,)
            ),
            slice_sizes=(1,),
            mode=lax.GatherScatterMode.PROMISE_IN_BOUNDS,
        )

    tm1 = _lane_gather(tmvec, jnp.minimum(rowv + 7, 15))
    tp1 = _lane_gather(t0vec, jnp.minimum(rowv + 1, 15))
    tvec = [tm1, t0vec, tp1]
    flat = [rowv * NUM_CLASSES + tvec[d] for d in range(3)]

    on1v = jnp.full((16,), ON1, jnp.float32)
    plsc.store_scatter(blk_v, [flat[1]], on1v, mask=lane_ok)
    plsc.store_scatter(blk_v, [flat[2]], on1v, mask=lane_ok)
    pltpu.sync_copy(blk_v, o1_ref.at[pl.ds(base * NUM_CLASSES, RPW * NUM_CLASSES)])

    plsc.store_scatter(blk_v, [flat[1]], offv, mask=lane_ok)
    plsc.store_scatter(blk_v, [flat[2]], offv, mask=lane_ok)
    for d in range(3):
        plsc.store_scatter(
            blk_v, [flat[d]], jnp.full((16,), float(ON2[d]), jnp.float32),
            mask=lane_ok,
        )
    pltpu.sync_copy(blk_v, o2_ref.at[pl.ds(base * NUM_CLASSES, RPW * NUM_CLASSES)])


_onehot_sc = functools.partial(
    pl.kernel,
    out_type=[
        jax.ShapeDtypeStruct((B * NUM_CLASSES,), jnp.float32),
        jax.ShapeDtypeStruct((B * NUM_CLASSES,), jnp.float32),
    ],
    mesh=plsc.VectorSubcoreMesh(core_axis_name="c", subcore_axis_name="s"),
    compiler_params=pltpu.CompilerParams(needs_layout_passes=False),
    scratch_types=[
        pltpu.VMEM((8 + B + 16,), jnp.int32),
        pltpu.VMEM((RPW * NUM_CLASSES,), jnp.float32),
    ],
)(_onehot_sc_body)


def kernel(x, target):
    b, c, h, w = x.shape
    n = c * h * w
    sub, lane = n // 128, 128
    xf = x.reshape(b, sub, lane)
    mask = jnp.asarray(
        np.tile(_PATCH_MASK.reshape(-1), c).reshape(1, sub, lane)
    )

    x_out = pl.pallas_call(
        _mix_body,
        grid=(b // ROWS,),
        in_specs=[
            pl.BlockSpec((ROWS, sub, lane), lambda t: (t, 0, 0)),
            pl.BlockSpec((1, sub, lane), lambda t: (((t + 1) * ROWS) % b, 0, 0)),
            pl.BlockSpec((1, sub, lane), lambda t: (0, 0, 0)),
        ],
        out_specs=pl.BlockSpec((ROWS, sub, lane), lambda t: (t, 0, 0)),
        out_shape=jax.ShapeDtypeStruct((b, sub, lane), x.dtype),
        compiler_params=pltpu.CompilerParams(vmem_limit_bytes=128 * 1024 * 1024),
    )(xf, xf, mask).reshape(b, c, h, w)

    m2o_flat, m2m_flat = _onehot_sc(target.astype(jnp.int32))
    m2o_target = m2o_flat.reshape(b, NUM_CLASSES)
    m2m_target = m2m_flat.reshape(b, NUM_CLASSES)

    return (x_out, m2o_target, m2m_target)


# speedup vs baseline: 1.0141x; 1.0000x over previous
import functools

import jax
import jax.numpy as jnp
import numpy as np
from jax import lax
from jax.experimental import pallas as pl
from jax.experimental.pallas import tpu as pltpu
from jax.experimental.pallas import tpu_sc as plsc

NUM_CLASSES = 4096
MIX_NUM = 2
PATCH = 16
SMOOTH = 0.1
B = 256
OFF = SMOOTH / NUM_CLASSES
ON1 = (1.0 - SMOOTH) / MIX_NUM + OFF
ON2 = ((1.0 - SMOOTH) * np.array([0.5, 1.0, 0.5]) / MIX_NUM + OFF).astype(np.float32)


_SEL_BITS = (
    "1100000001001110010001010111100001001010110000111001100101011010"
    "1010100101111001000010111101010101101001111101000101100110101111"
    "1001111110000101011001110010001011101110101110000000111000010011"
    "0110"
)


def _compute_patch_mask():
    sel = np.array([int(ch) for ch in _SEL_BITS]).reshape(14, 14)
    mask = np.repeat(np.repeat(sel, PATCH, 0), PATCH, 1)
    return mask.astype(np.float32)


_PATCH_MASK = _compute_patch_mask()


ROWS = 16


def _mix_body(xa_ref, xb_ref, mask_ref, o_ref):
    m = mask_ref[...] != 0.0
    o_ref[0 : ROWS - 1] = jnp.where(m, xa_ref[1:ROWS], xa_ref[0 : ROWS - 1])
    o_ref[ROWS - 1 : ROWS] = jnp.where(m, xb_ref[...], xa_ref[ROWS - 1 : ROWS])


RPW = B // 32


def _onehot_sc_body(t_ref, o1_ref, o2_ref, t_v, blk_v):
    cid = lax.axis_index("c")
    sid = lax.axis_index("s")
    wid = sid * 2 + cid
    base = wid * RPW

    rowv = jnp.arange(16, dtype=jnp.int32)
    pltpu.sync_copy(t_ref, t_v.at[pl.ds(8, B)])
    t_v[pl.ds(8 + B, 16)] = t_v[pl.ds(8, 16)]
    plsc.store_scatter(t_v, [rowv], t_v[pl.ds(B, 16)], mask=rowv < 8)

    offv = jnp.full((16,), OFF, jnp.float32)

    def fill_body(i, carry):
        o = i * 256
        for k in range(16):
            blk_v[pl.ds(o + k * 16, 16)] = offv
        return carry

    lax.fori_loop(0, RPW * NUM_CLASSES // 256, fill_body, None)
    lane_ok = rowv < RPW
    t0vec = t_v[pl.ds(base + 8, 16)]
    tmvec = t_v[pl.ds(base, 16)]
    def _lane_gather(vec, idx):
        return lax.gather(
            vec,
            idx[:, None],
            lax.GatherDimensionNumbers(
                offset_dims=(), collapsed_slice_dims=(0,), start_index_map=(0,)
            ),
            slice_sizes=(1,),
            mode=lax.GatherScatterMode.PROMISE_IN_BOUNDS,
        )

    tm1 = _lane_gather(tmvec, jnp.minimum(rowv + 7, 15))
    tp1 = _lane_gather(t0vec, jnp.minimum(rowv + 1, 15))
    tvec = [tm1, t0vec, tp1]
    flat = [rowv * NUM_CLASSES + tvec[d] for d in range(3)]

    on1v = jnp.full((16,), ON1, jnp.float32)
    plsc.store_scatter(blk_v, [flat[1]], on1v, mask=lane_ok)
    plsc.store_scatter(blk_v, [flat[2]], on1v, mask=lane_ok)
    pltpu.sync_copy(blk_v, o1_ref.at[pl.ds(base * NUM_CLASSES, RPW * NUM_CLASSES)])

    plsc.store_scatter(blk_v, [flat[1]], offv, mask=lane_ok)
    plsc.store_scatter(blk_v, [flat[2]], offv, mask=lane_ok)
    for d in range(3):
        plsc.store_scatter(
            blk_v, [flat[d]], jnp.full((16,), float(ON2[d]), jnp.float32),
            mask=lane_ok,
        )
    pltpu.sync_copy(blk_v, o2_ref.at[pl.ds(base * NUM_CLASSES, RPW * NUM_CLASSES)])


_onehot_sc = functools.partial(
    pl.kernel,
    out_type=[
        jax.ShapeDtypeStruct((B * NUM_CLASSES,), jnp.float32),
        jax.ShapeDtypeStruct((B * NUM_CLASSES,), jnp.float32),
    ],
    mesh=plsc.VectorSubcoreMesh(core_axis_name="c", subcore_axis_name="s"),
    compiler_params=pltpu.CompilerParams(needs_layout_passes=False),
    scratch_types=[
        pltpu.VMEM((8 + B + 16,), jnp.int32),
        pltpu.VMEM((RPW * NUM_CLASSES,), jnp.float32),
    ],
)(_onehot_sc_body)


def kernel(x, target):
    b, c, h, w = x.shape
    n = c * h * w
    sub, lane = n // 128, 128
    xf = x.reshape(b, sub, lane)
    mask = jnp.asarray(
        np.tile(_PATCH_MASK.reshape(-1), c).reshape(1, sub, lane)
    )

    x_out = pl.pallas_call(
        _mix_body,
        grid=(b // ROWS,),
        in_specs=[
            pl.BlockSpec((ROWS, sub, lane), lambda t: (t, 0, 0)),
            pl.BlockSpec((1, sub, lane), lambda t: (((t + 1) * ROWS) % b, 0, 0)),
            pl.BlockSpec((1, sub, lane), lambda t: (0, 0, 0)),
        ],
        out_specs=pl.BlockSpec((ROWS, sub, lane), lambda t: (t, 0, 0)),
        out_shape=jax.ShapeDtypeStruct((b, sub, lane), x.dtype),
        compiler_params=pltpu.CompilerParams(vmem_limit_bytes=128 * 1024 * 1024),
    )(xf, xf, mask).reshape(b, c, h, w)

    m2o_flat, m2m_flat = _onehot_sc(target.astype(jnp.int32))
    m2o_target = m2o_flat.reshape(b, NUM_CLASSES)
    m2m_target = m2m_flat.reshape(b, NUM_CLASSES)

    return (x_out, m2o_target, m2m_target)


# speedup vs baseline: 1.0147x; 1.0006x over previous
import functools

import jax
import jax.numpy as jnp
import numpy as np
from jax import lax
from jax.experimental import pallas as pl
from jax.experimental.pallas import tpu as pltpu
from jax.experimental.pallas import tpu_sc as plsc

NUM_CLASSES = 4096
MIX_NUM = 2
PATCH = 16
SMOOTH = 0.1
B = 256
OFF = SMOOTH / NUM_CLASSES
ON1 = (1.0 - SMOOTH) / MIX_NUM + OFF
ON2 = ((1.0 - SMOOTH) * np.array([0.5, 1.0, 0.5]) / MIX_NUM + OFF).astype(np.float32)


_SEL_BITS = (
    "1100000001001110010001010111100001001010110000111001100101011010"
    "1010100101111001000010111101010101101001111101000101100110101111"
    "1001111110000101011001110010001011101110101110000000111000010011"
    "0110"
)


def _compute_patch_mask():
    sel = np.array([int(ch) for ch in _SEL_BITS]).reshape(14, 14)
    mask = np.repeat(np.repeat(sel, PATCH, 0), PATCH, 1)
    return mask.astype(np.float32)


_PATCH_MASK = _compute_patch_mask()


ROWS = 16


def _mix_body(xa_ref, xb_ref, mask_ref, o_ref):
    m = mask_ref[...] != 0.0
    o_ref[0 : ROWS - 1] = jnp.where(m, xa_ref[1:ROWS], xa_ref[0 : ROWS - 1])
    o_ref[ROWS - 1 : ROWS] = jnp.where(m, xb_ref[...], xa_ref[ROWS - 1 : ROWS])


RPW = B // 32


def _onehot_sc_body(t_ref, o1_ref, o2_ref, t_v, blk_v):
    cid = lax.axis_index("c")
    sid = lax.axis_index("s")
    wid = sid * 2 + cid
    base = wid * RPW

    rowv = jnp.arange(16, dtype=jnp.int32)
    pltpu.sync_copy(t_ref, t_v.at[pl.ds(8, B)])
    t_v[pl.ds(8 + B, 16)] = t_v[pl.ds(8, 16)]
    plsc.store_scatter(t_v, [rowv], t_v[pl.ds(B, 16)], mask=rowv < 8)

    offv = jnp.full((16,), OFF, jnp.float32)

    def fill_body(i, carry):
        o = i * 256
        for k in range(16):
            blk_v[pl.ds(o + k * 16, 16)] = offv
        return carry

    lax.fori_loop(0, RPW * NUM_CLASSES // 256, fill_body, None)
    lane_ok = rowv < RPW
    t0vec = t_v[pl.ds(base + 8, 16)]
    tmvec = t_v[pl.ds(base, 16)]
    def _lane_gather(vec, idx):
        return lax.gather(
            vec,
            idx[:, None],
            lax.GatherDimensionNumbers(
                offset_dims=(), collapsed_slice_dims=(0,), start_index_map=(0,)
            ),
            slice_sizes=(1,),
            mode=lax.GatherScatterMode.PROMISE_IN_BOUNDS,
        )

    tm1 = _lane_gather(tmvec, jnp.minimum(rowv + 7, 15))
    tp1 = _lane_gather(t0vec, jnp.minimum(rowv + 1, 15))
    tvec = [tm1, t0vec, tp1]
    flat = [rowv * NUM_CLASSES + tvec[d] for d in range(3)]

    on1v = jnp.full((16,), ON1, jnp.float32)
    plsc.store_scatter(blk_v, [flat[1]], on1v, mask=lane_ok)
    plsc.store_scatter(blk_v, [flat[2]], on1v, mask=lane_ok)
    pltpu.sync_copy(blk_v, o1_ref.at[pl.ds(base * NUM_CLASSES, RPW * NUM_CLASSES)])

    for d in range(3):
        plsc.store_scatter(
            blk_v, [flat[d]], jnp.full((16,), float(ON2[d]), jnp.float32),
            mask=lane_ok,
        )
    pltpu.sync_copy(blk_v, o2_ref.at[pl.ds(base * NUM_CLASSES, RPW * NUM_CLASSES)])


_onehot_sc = functools.partial(
    pl.kernel,
    out_type=[
        jax.ShapeDtypeStruct((B * NUM_CLASSES,), jnp.float32),
        jax.ShapeDtypeStruct((B * NUM_CLASSES,), jnp.float32),
    ],
    mesh=plsc.VectorSubcoreMesh(core_axis_name="c", subcore_axis_name="s"),
    compiler_params=pltpu.CompilerParams(needs_layout_passes=False),
    scratch_types=[
        pltpu.VMEM((8 + B + 16,), jnp.int32),
        pltpu.VMEM((RPW * NUM_CLASSES,), jnp.float32),
    ],
)(_onehot_sc_body)


def kernel(x, target):
    b, c, h, w = x.shape
    n = c * h * w
    sub, lane = n // 128, 128
    xf = x.reshape(b, sub, lane)
    mask = jnp.asarray(
        np.tile(_PATCH_MASK.reshape(-1), c).reshape(1, sub, lane)
    )

    x_out = pl.pallas_call(
        _mix_body,
        grid=(b // ROWS,),
        in_specs=[
            pl.BlockSpec((ROWS, sub, lane), lambda t: (t, 0, 0)),
            pl.BlockSpec((1, sub, lane), lambda t: (((t + 1) * ROWS) % b, 0, 0)),
            pl.BlockSpec((1, sub, lane), lambda t: (0, 0, 0)),
        ],
        out_specs=pl.BlockSpec((ROWS, sub, lane), lambda t: (t, 0, 0)),
        out_shape=jax.ShapeDtypeStruct((b, sub, lane), x.dtype),
        compiler_params=pltpu.CompilerParams(vmem_limit_bytes=128 * 1024 * 1024),
    )(xf, xf, mask).reshape(b, c, h, w)

    m2o_flat, m2m_flat = _onehot_sc(target.astype(jnp.int32))
    m2o_target = m2o_flat.reshape(b, NUM_CLASSES)
    m2m_target = m2m_flat.reshape(b, NUM_CLASSES)

    return (x_out, m2o_target, m2m_target)
